# Initial kernel scaffold; baseline (speedup 1.0000x reference)
import jax
import jax.numpy as jnp
from jax.experimental import pallas as pl


def kernel(signal, bc, W_down, b_down, gamma_down, beta_down, templates_0, bias_0, gamma_0, beta_0, templates_1, bias_1, gamma_1, beta_1, templates_2, bias_2, gamma_2, beta_2, W_out, b_out):
    raise NotImplementedError("write your pallas kernel here")


# speedup vs baseline: 3.0301x; 3.0301x over previous
import functools

import numpy as np
import jax
import jax.numpy as jnp
from jax import lax
from jax.experimental import pallas as pl
from jax.experimental.pallas import tpu as pltpu
from jax.experimental.pallas import tpu_sc as plsc

NV = 6890
NPAD = 6912
RR, AA = 3, 6
RA = RR * AA
MPAD = NPAD * RA
NW = 32
ROWS_W = MPAD // NW
CHUNK = 128
NFULL = ROWS_W // CHUNK
REM = ROWS_W - NFULL * CHUNK
INV_S = float(1.0 / np.sqrt(1.0 + 1e-3))

SIG_D = 544
DOWN_D = 64
KP = 128
NROT = 3


def _make_gather(D):
    mesh = plsc.VectorSubcoreMesh(core_axis_name="c", subcore_axis_name="s")

    def body(x_hbm, i0h, i1h, i2h, w0h, w1h, w2h, out_hbm,
             i0, i1, i2, w0, w1, w2, g0, g1, g2, ov, sem):
        wid = lax.axis_index("s") * 2 + lax.axis_index("c")
        base = wid * ROWS_W
        pltpu.sync_copy(i0h.at[pl.ds(base, ROWS_W)], i0)
        pltpu.sync_copy(i1h.at[pl.ds(base, ROWS_W)], i1)
        pltpu.sync_copy(i2h.at[pl.ds(base, ROWS_W)], i2)
        pltpu.sync_copy(w0h.at[pl.ds(base, ROWS_W)], w0)
        pltpu.sync_copy(w1h.at[pl.ds(base, ROWS_W)], w1)
        pltpu.sync_copy(w2h.at[pl.ds(base, ROWS_W)], w2)

        def do_chunk(start, n):
            c0 = pltpu.async_copy(
                x_hbm.at[i0.at[pl.ds(start, n)]], g0.at[pl.ds(0, n)], sem)
            c1 = pltpu.async_copy(
                x_hbm.at[i1.at[pl.ds(start, n)]], g1.at[pl.ds(0, n)], sem)
            c2 = pltpu.async_copy(
                x_hbm.at[i2.at[pl.ds(start, n)]], g2.at[pl.ds(0, n)], sem)
            c0.wait()
            c1.wait()
            c2.wait()

            def grp(q, carry):
                rbase = q * 16
                wv0 = w0[pl.ds(start + rbase, 16)]
                wv1 = w1[pl.ds(start + rbase, 16)]
                wv2 = w2[pl.ds(start + rbase, 16)]
                for i in range(16):
                    a, b, cc = wv0[i], wv1[i], wv2[i]
                    r = rbase + i
                    for db in range(D // 16):
                        s = pl.ds(db * 16, 16)
                        ov[r, s] = g0[r, s] * a + g1[r, s] * b + g2[r, s] * cc
                return carry

            lax.fori_loop(0, n // 16, grp, 0)
            pltpu.sync_copy(ov.at[pl.ds(0, n)],
                            out_hbm.at[pl.ds(base + start, n)])

        def full_chunk(g, carry):
            do_chunk(pl.multiple_of(g * CHUNK, CHUNK), CHUNK)
            return carry

        lax.fori_loop(0, NFULL, full_chunk, 0)
        do_chunk(NFULL * CHUNK, REM)

    return pl.kernel(
        body, mesh=mesh,
        out_type=jax.ShapeDtypeStruct((MPAD, D), jnp.float32),
        scratch_types=[
            pltpu.VMEM((ROWS_W,), jnp.int32),
            pltpu.VMEM((ROWS_W,), jnp.int32),
            pltpu.VMEM((ROWS_W,), jnp.int32),
            pltpu.VMEM((ROWS_W,), jnp.float32),
            pltpu.VMEM((ROWS_W,), jnp.float32),
            pltpu.VMEM((ROWS_W,), jnp.float32),
            pltpu.VMEM((CHUNK, D), jnp.float32),
            pltpu.VMEM((CHUNK, D), jnp.float32),
            pltpu.VMEM((CHUNK, D), jnp.float32),
            pltpu.VMEM((CHUNK, D), jnp.float32),
            pltpu.SemaphoreType.DMA,
        ],
    )


_gather128 = _make_gather(KP)


def _down_proj(signal_p, Wd, b, s, bt):
    BM = 576

    def body(a_ref, w_ref, b_ref, s_ref, bt_ref, o_ref):
        y = jnp.dot(a_ref[...], w_ref[...], preferred_element_type=jnp.float32)
        y = jnp.maximum(y + b_ref[...], 0.0)
        o_ref[...] = y * s_ref[...] + bt_ref[...]

    return pl.pallas_call(
        body,
        grid=(NPAD // BM,),
        in_specs=[
            pl.BlockSpec((BM, SIG_D), lambda i: (i, 0)),
            pl.BlockSpec((SIG_D, KP), lambda i: (0, 0)),
            pl.BlockSpec((1, KP), lambda i: (0, 0)),
            pl.BlockSpec((1, KP), lambda i: (0, 0)),
            pl.BlockSpec((1, KP), lambda i: (0, 0)),
        ],
        out_specs=pl.BlockSpec((BM, KP), lambda i: (i, 0)),
        out_shape=jax.ShapeDtypeStruct((NPAD, KP), jnp.float32),
    )(signal_p, Wd, b, s, bt)


def _conv_layer(interp2, Tm, btile, s, bt):
    BM = 576
    Kin = interp2.shape[1]

    def body(a_ref, t_ref, b_ref, s_ref, bt_ref, o_ref):
        y = jnp.dot(a_ref[...], t_ref[...], preferred_element_type=jnp.float32)
        y = jnp.maximum(y + b_ref[...], 0.0)
        m = jnp.maximum(jnp.maximum(y[:, :KP], y[:, KP:2 * KP]),
                        y[:, 2 * KP:3 * KP])
        o_ref[...] = m * s_ref[...] + bt_ref[...]

    return pl.pallas_call(
        body,
        grid=(NPAD // BM,),
        in_specs=[
            pl.BlockSpec((BM, Kin), lambda i: (i, 0)),
            pl.BlockSpec((Kin, NROT * KP), lambda i: (0, 0)),
            pl.BlockSpec((1, NROT * KP), lambda i: (0, 0)),
            pl.BlockSpec((1, KP), lambda i: (0, 0)),
            pl.BlockSpec((1, KP), lambda i: (0, 0)),
        ],
        out_specs=pl.BlockSpec((BM, KP), lambda i: (i, 0)),
        out_shape=jax.ShapeDtypeStruct((NPAD, KP), jnp.float32),
    )(interp2, Tm, btile, s, bt)


def _final_proj(x, Wo, bo):
    BM, BN = 512, 1024

    def body(a_ref, w_ref, b_ref, o_ref):
        o_ref[...] = jnp.dot(a_ref[...], w_ref[...],
                             preferred_element_type=jnp.float32) + b_ref[...]

    return pl.pallas_call(
        body,
        grid=(pl.cdiv(NV, BM), pl.cdiv(NV, BN)),
        in_specs=[
            pl.BlockSpec((BM, KP), lambda i, j: (i, 0)),
            pl.BlockSpec((KP, BN), lambda i, j: (0, j)),
            pl.BlockSpec((1, BN), lambda i, j: (0, j)),
        ],
        out_specs=pl.BlockSpec((BM, BN), lambda i, j: (i, j)),
        out_shape=jax.ShapeDtypeStruct((NV, NV), jnp.float32),
    )(x, Wo, bo)


def _rot_templates(tpl, rd, kp, dp):
    K, Rq, Aq, D = tpl.shape
    tpl = jnp.pad(tpl, ((0, kp - K), (0, 0), (0, 0), (0, dp - D)))
    mats = []
    for rot in range(0, Aq, rd):
        t = jnp.roll(tpl, -rot, axis=2)
        mats.append(t.transpose(1, 2, 3, 0).reshape(Rq * Aq * dp, kp))
    return jnp.concatenate(mats, axis=1)


def _pad1(v, n):
    return jnp.pad(v, (0, n - v.shape[0]))


def kernel(signal, bc, W_down, b_down, gamma_down, beta_down,
           templates_0, bias_0, gamma_0, beta_0,
           templates_1, bias_1, gamma_1, beta_1,
           templates_2, bias_2, gamma_2, beta_2,
           W_out, b_out):
    f32 = jnp.float32
    bc_idx = bc[..., 0].astype(jnp.int32).reshape(NV * RA, 3)
    bc_w = bc[..., 1].reshape(NV * RA, 3)
    pad_m = MPAD - NV * RA
    bc_idx = jnp.pad(bc_idx, ((0, pad_m), (0, 0)))
    bc_w = jnp.pad(bc_w, ((0, pad_m), (0, 0)))
    i0, i1, i2 = bc_idx[:, 0], bc_idx[:, 1], bc_idx[:, 2]
    w0, w1, w2 = bc_w[:, 0], bc_w[:, 1], bc_w[:, 2]

    signal_p = jnp.pad(signal, ((0, NPAD - NV), (0, 0)))
    x = _down_proj(signal_p, jnp.pad(W_down, ((0, 0), (0, KP - DOWN_D))),
                   _pad1(b_down, KP).reshape(1, KP),
                   (_pad1(gamma_down, KP) * INV_S).reshape(1, KP),
                   _pad1(beta_down, KP).reshape(1, KP))

    layers = (
        (templates_0, bias_0, gamma_0, beta_0, 2),
        (templates_1, bias_1, gamma_1, beta_1, 2),
        (templates_2, bias_2, gamma_2, beta_2, 2),
    )
    for tpl, b, g, bt, rd in layers:
        Tm = _rot_templates(tpl, rd, KP, KP)
        btile = jnp.tile(_pad1(b, KP), NROT).reshape(1, NROT * KP)
        interp = _gather128(x, i0, i1, i2, w0, w1, w2)
        interp2 = interp.reshape(NPAD, RA * KP)
        x = _conv_layer(interp2, Tm, btile,
                        (_pad1(g, KP) * INV_S).reshape(1, KP),
                        _pad1(bt, KP).reshape(1, KP))

    return _final_proj(x, W_out, b_out.reshape(1, NV))


# speedup vs baseline: 3.4081x; 1.1247x over previous
import functools

import numpy as np
import jax
import jax.numpy as jnp
from jax import lax
from jax.experimental import pallas as pl
from jax.experimental.pallas import tpu as pltpu
from jax.experimental.pallas import tpu_sc as plsc

NV = 6890
NPAD = 6912
RR, AA = 3, 6
RA = RR * AA
MPAD = NPAD * RA
NW = 32
ROWS_W = MPAD // NW
CHUNK = 48
NCH = ROWS_W // CHUNK
RING = 3
OUTER = NCH // RING
INV_S = float(1.0 / np.sqrt(1.0 + 1e-3))

SIG_D = 544
DOWN_D = 64
KP = 128
NROT = 3


def _make_gather(D):
    mesh = plsc.VectorSubcoreMesh(core_axis_name="c", subcore_axis_name="s")

    def body(x_hbm, i0h, i1h, i2h, w0h, w1h, w2h, out_hbm, *refs):
        (i0, i1, i2, w0, w1, w2,
         ga0, gb0, gc0, ga1, gb1, gc1, ga2, gb2, gc2,
         gs0, gs1, gs2, os0, os1, os2) = refs
        G = ((ga0, gb0, gc0), (ga1, gb1, gc1), (ga2, gb2, gc2))
        GS = (gs0, gs1, gs2)
        OS = (os0, os1, os2)
        idx = (i0, i1, i2)
        wts = (w0, w1, w2)
        wid = lax.axis_index("s") * 2 + lax.axis_index("c")
        base = wid * ROWS_W
        for h, v in ((i0h, i0), (i1h, i1), (i2h, i2),
                     (w0h, w0), (w1h, w1), (w2h, w2)):
            pltpu.sync_copy(h.at[pl.ds(base, ROWS_W)], v)

        def issue_gather(c, s):
            start = c * CHUNK
            for j in range(3):
                pltpu.async_copy(
                    x_hbm.at[idx[j].at[pl.ds(start, CHUNK)]], G[s][j], GS[s])

        def wait_gather(s):
            for j in range(3):
                pltpu.make_async_copy(
                    out_hbm.at[pl.ds(0, CHUNK)], G[s][j], GS[s]).wait()

        def issue_out(c, s):
            pltpu.async_copy(
                G[s][0], out_hbm.at[pl.ds(base + c * CHUNK, CHUNK)], OS[s])

        def wait_out(s):
            pltpu.make_async_copy(
                G[s][0], out_hbm.at[pl.ds(0, CHUNK)], OS[s]).wait()

        def compute(c, s):
            ga, gb, gc = G[s]
            start = c * CHUNK

            def grp(q, carry):
                rbase = q * 16
                wv0 = wts[0][pl.ds(start + rbase, 16)]
                wv1 = wts[1][pl.ds(start + rbase, 16)]
                wv2 = wts[2][pl.ds(start + rbase, 16)]
                for i in range(16):
                    a, b2, c2 = wv0[i], wv1[i], wv2[i]
                    for db in range(D // 16):
                        sdb = pl.ds(db * 16, 16)
                        ga[rbase + i, sdb] = (ga[rbase + i, sdb] * a
                                              + gb[rbase + i, sdb] * b2
                                              + gc[rbase + i, sdb] * c2)
                return carry

            lax.fori_loop(0, CHUNK // 16, grp, 0)

        issue_gather(0, 0)
        issue_gather(1, 1)

        def outer(k, carry):
            for b in range(RING):
                c = 3 * k + b
                sn = (b + 2) % 3
                if b == 0:
                    @pl.when(k >= 1)
                    def _():
                        wait_out(sn)
                    issue_gather(c + 2, sn)
                else:
                    wait_out(sn)

                    @pl.when(k <= OUTER - 2)
                    def _():
                        issue_gather(c + 2, sn)
                wait_gather(b)
                compute(c, b)
                issue_out(c, b)
            return carry

        lax.fori_loop(0, OUTER, outer, 0)
        wait_out(2)

    return pl.kernel(
        body, mesh=mesh,
        out_type=jax.ShapeDtypeStruct((MPAD, D), jnp.float32),
        scratch_types=(
            [pltpu.VMEM((ROWS_W,), jnp.int32) for _ in range(3)]
            + [pltpu.VMEM((ROWS_W,), jnp.float32) for _ in range(3)]
            + [pltpu.VMEM((CHUNK, D), jnp.float32) for _ in range(9)]
            + [pltpu.SemaphoreType.DMA for _ in range(6)]
        ),
    )


_gather128 = _make_gather(KP)


def _down_proj(signal_p, Wd, b, s, bt):
    BM = 576

    def body(a_ref, w_ref, b_ref, s_ref, bt_ref, o_ref):
        y = jnp.dot(a_ref[...], w_ref[...], preferred_element_type=jnp.float32)
        y = jnp.maximum(y + b_ref[...], 0.0)
        o_ref[...] = y * s_ref[...] + bt_ref[...]

    return pl.pallas_call(
        body,
        grid=(NPAD // BM,),
        in_specs=[
            pl.BlockSpec((BM, SIG_D), lambda i: (i, 0)),
            pl.BlockSpec((SIG_D, KP), lambda i: (0, 0)),
            pl.BlockSpec((1, KP), lambda i: (0, 0)),
            pl.BlockSpec((1, KP), lambda i: (0, 0)),
            pl.BlockSpec((1, KP), lambda i: (0, 0)),
        ],
        out_specs=pl.BlockSpec((BM, KP), lambda i: (i, 0)),
        out_shape=jax.ShapeDtypeStruct((NPAD, KP), jnp.float32),
    )(signal_p, Wd, b, s, bt)


def _conv_layer(interp3, Tm, btile, s, bt):
    BM = 576

    def body(a_ref, t_ref, b_ref, s_ref, bt_ref, o_ref, acc):
        ra = pl.program_id(1)

        @pl.when(ra == 0)
        def _():
            acc[...] = jnp.broadcast_to(b_ref[...], (BM, NROT * KP))

        acc[...] += jnp.dot(a_ref[0], t_ref[...],
                            preferred_element_type=jnp.float32)

        @pl.when(ra == RA - 1)
        def _():
            y = jnp.maximum(acc[...], 0.0)
            m = jnp.maximum(jnp.maximum(y[:, :KP], y[:, KP:2 * KP]),
                            y[:, 2 * KP:3 * KP])
            o_ref[...] = m * s_ref[...] + bt_ref[...]

    return pl.pallas_call(
        body,
        grid=(NPAD // BM, RA),
        in_specs=[
            pl.BlockSpec((1, BM, KP), lambda i, ra: (ra, i, 0)),
            pl.BlockSpec((KP, NROT * KP), lambda i, ra: (ra, 0)),
            pl.BlockSpec((1, NROT * KP), lambda i, ra: (0, 0)),
            pl.BlockSpec((1, KP), lambda i, ra: (0, 0)),
            pl.BlockSpec((1, KP), lambda i, ra: (0, 0)),
        ],
        out_specs=pl.BlockSpec((BM, KP), lambda i, ra: (i, 0)),
        out_shape=jax.ShapeDtypeStruct((NPAD, KP), jnp.float32),
        scratch_shapes=[pltpu.VMEM((BM, NROT * KP), jnp.float32)],
    )(interp3, Tm, btile, s, bt)


def _final_proj(x, Wo, bo):
    BM, BN = 512, 1024

    def body(a_ref, w_ref, b_ref, o_ref):
        o_ref[...] = jnp.dot(a_ref[...], w_ref[...],
                             preferred_element_type=jnp.float32) + b_ref[...]

    return pl.pallas_call(
        body,
        grid=(pl.cdiv(NV, BM), pl.cdiv(NV, BN)),
        in_specs=[
            pl.BlockSpec((BM, KP), lambda i, j: (i, 0)),
            pl.BlockSpec((KP, BN), lambda i, j: (0, j)),
            pl.BlockSpec((1, BN), lambda i, j: (0, j)),
        ],
        out_specs=pl.BlockSpec((BM, BN), lambda i, j: (i, j)),
        out_shape=jax.ShapeDtypeStruct((NV, NV), jnp.float32),
    )(x, Wo, bo)


def _rot_templates(tpl, rd, kp, dp):
    K, Rq, Aq, D = tpl.shape
    tpl = jnp.pad(tpl, ((0, kp - K), (0, 0), (0, 0), (0, dp - D)))
    mats = []
    for rot in range(0, Aq, rd):
        t = jnp.roll(tpl, -rot, axis=2)
        mats.append(t.transpose(1, 2, 3, 0).reshape(Rq * Aq * dp, kp))
    return jnp.concatenate(mats, axis=1)


def _pad1(v, n):
    return jnp.pad(v, (0, n - v.shape[0]))


def kernel(signal, bc, W_down, b_down, gamma_down, beta_down,
           templates_0, bias_0, gamma_0, beta_0,
           templates_1, bias_1, gamma_1, beta_1,
           templates_2, bias_2, gamma_2, beta_2,
           W_out, b_out):
    bc_idx = bc[..., 0].astype(jnp.int32).reshape(NV, RA, 3)
    bc_w = bc[..., 1].reshape(NV, RA, 3)
    bc_idx = jnp.pad(bc_idx.transpose(1, 0, 2), ((0, 0), (0, NPAD - NV), (0, 0)))
    bc_w = jnp.pad(bc_w.transpose(1, 0, 2), ((0, 0), (0, NPAD - NV), (0, 0)))
    bc_idx = bc_idx.reshape(MPAD, 3)
    bc_w = bc_w.reshape(MPAD, 3)
    i0, i1, i2 = bc_idx[:, 0], bc_idx[:, 1], bc_idx[:, 2]
    w0, w1, w2 = bc_w[:, 0], bc_w[:, 1], bc_w[:, 2]

    signal_p = jnp.pad(signal, ((0, NPAD - NV), (0, 0)))
    x = _down_proj(signal_p, jnp.pad(W_down, ((0, 0), (0, KP - DOWN_D))),
                   _pad1(b_down, KP).reshape(1, KP),
                   (_pad1(gamma_down, KP) * INV_S).reshape(1, KP),
                   _pad1(beta_down, KP).reshape(1, KP))

    layers = (
        (templates_0, bias_0, gamma_0, beta_0, 2),
        (templates_1, bias_1, gamma_1, beta_1, 2),
        (templates_2, bias_2, gamma_2, beta_2, 2),
    )
    for tpl, b, g, bt, rd in layers:
        Tm = _rot_templates(tpl, rd, KP, KP)
        btile = jnp.tile(_pad1(b, KP), NROT).reshape(1, NROT * KP)
        interp = _gather128(x, i0, i1, i2, w0, w1, w2)
        interp3 = interp.reshape(RA, NPAD, KP)
        x = _conv_layer(interp3, Tm, btile,
                        (_pad1(g, KP) * INV_S).reshape(1, KP),
                        _pad1(bt, KP).reshape(1, KP))

    return _final_proj(x, W_out, b_out.reshape(1, NV))


# speedup vs baseline: 3.4213x; 1.0039x over previous
import functools

import numpy as np
import jax
import jax.numpy as jnp
from jax import lax
from jax.experimental import pallas as pl
from jax.experimental.pallas import tpu as pltpu
from jax.experimental.pallas import tpu_sc as plsc

NV = 6890
NPAD = 6912
RR, AA = 3, 6
RA = RR * AA
MPAD = NPAD * RA
NW = 32
ROWS_W = MPAD // NW
CHUNK = 48
NCH = ROWS_W // CHUNK
RING = 3
OUTER = NCH // RING
INV_S = float(1.0 / np.sqrt(1.0 + 1e-3))

SIG_D = 544
DOWN_D = 64
KP = 128
NROT = 3


def _make_gather(D):
    mesh = plsc.VectorSubcoreMesh(core_axis_name="c", subcore_axis_name="s")

    def body(x_hbm, i0h, i1h, i2h, w0h, w1h, w2h, out_hbm, *refs):
        (i0, i1, i2, w0, w1, w2,
         ga0, gb0, gc0, ga1, gb1, gc1, ga2, gb2, gc2,
         ov0, ov1, ov2,
         gs0, gs1, gs2, os0, os1, os2) = refs
        ov = (ov0, ov1, ov2)
        G = ((ga0, gb0, gc0), (ga1, gb1, gc1), (ga2, gb2, gc2))
        GS = (gs0, gs1, gs2)
        OS = (os0, os1, os2)
        idx = (i0, i1, i2)
        wts = (w0, w1, w2)
        wid = lax.axis_index("s") * 2 + lax.axis_index("c")
        base = wid * ROWS_W
        for h, v in ((i0h, i0), (i1h, i1), (i2h, i2),
                     (w0h, w0), (w1h, w1), (w2h, w2)):
            pltpu.sync_copy(h.at[pl.ds(base, ROWS_W)], v)

        def issue_gather(c, s):
            start = c * CHUNK
            for j in range(3):
                pltpu.async_copy(
                    x_hbm.at[idx[j].at[pl.ds(start, CHUNK)]], G[s][j], GS[s])

        def wait_gather(s):
            for j in range(3):
                pltpu.make_async_copy(
                    x_hbm.at[pl.ds(0, CHUNK)], G[s][j], GS[s]).wait()

        def issue_out(c, s):
            pltpu.async_copy(
                ov[s], out_hbm.at[pl.ds(base + c * CHUNK, CHUNK)], OS[s])

        def wait_out(s):
            pltpu.make_async_copy(
                ov[s], out_hbm.at[pl.ds(0, CHUNK)], OS[s]).wait()

        def compute(c, s):
            ga, gb, gc = G[s]
            start = c * CHUNK

            def grp(q, carry):
                rbase = q * 16
                wv0 = wts[0][pl.ds(start + rbase, 16)]
                wv1 = wts[1][pl.ds(start + rbase, 16)]
                wv2 = wts[2][pl.ds(start + rbase, 16)]
                for i in range(16):
                    a, b2, c2 = wv0[i], wv1[i], wv2[i]
                    r = rbase + i
                    for db in range(D // 32):
                        sdb = pl.ds(db * 32, 32)
                        a0, a1 = plsc.unpack(ga[r, sdb], format=plsc.PackFormat.INTERLEAVED)
                        b0, b1 = plsc.unpack(gb[r, sdb], format=plsc.PackFormat.INTERLEAVED)
                        c0, c1 = plsc.unpack(gc[r, sdb], format=plsc.PackFormat.INTERLEAVED)
                        ov[s][r, pl.ds(db * 32, 16)] = (
                            a0 * a + b0 * b2 + c0 * c2)
                        ov[s][r, pl.ds(db * 32 + 16, 16)] = (
                            a1 * a + b1 * b2 + c1 * c2)
                return carry

            lax.fori_loop(0, CHUNK // 16, grp, 0)

        issue_gather(0, 0)
        issue_gather(1, 1)

        def outer(k, carry):
            for b in range(RING):
                c = 3 * k + b
                sn = (b + 2) % 3
                if b == 0:
                    @pl.when(k >= 1)
                    def _():
                        wait_out(sn)
                    issue_gather(c + 2, sn)
                else:
                    wait_out(sn)

                    @pl.when(k <= OUTER - 2)
                    def _():
                        issue_gather(c + 2, sn)
                wait_gather(b)
                compute(c, b)
                issue_out(c, b)
            return carry

        lax.fori_loop(0, OUTER, outer, 0)
        wait_out(2)

    return pl.kernel(
        body, mesh=mesh,
        out_type=jax.ShapeDtypeStruct((MPAD, D), jnp.float32),
        scratch_types=(
            [pltpu.VMEM((ROWS_W,), jnp.int32) for _ in range(3)]
            + [pltpu.VMEM((ROWS_W,), jnp.float32) for _ in range(3)]
            + [pltpu.VMEM((CHUNK, D), jnp.bfloat16) for _ in range(9)]
            + [pltpu.VMEM((CHUNK, D), jnp.float32) for _ in range(3)]
            + [pltpu.SemaphoreType.DMA for _ in range(6)]
        ),
        compiler_params=pltpu.CompilerParams(use_tc_tiling_on_sc=False, needs_layout_passes=False),
    )


_gather128 = _make_gather(KP)


def _down_proj(signal_p, Wd, b, s, bt):
    BM = 576

    def body(a_ref, w_ref, b_ref, s_ref, bt_ref, o_ref):
        y = jnp.dot(a_ref[...], w_ref[...], preferred_element_type=jnp.float32)
        y = jnp.maximum(y + b_ref[...], 0.0)
        o_ref[...] = (y * s_ref[...] + bt_ref[...]).astype(jnp.bfloat16)

    return pl.pallas_call(
        body,
        grid=(NPAD // BM,),
        in_specs=[
            pl.BlockSpec((BM, SIG_D), lambda i: (i, 0)),
            pl.BlockSpec((SIG_D, KP), lambda i: (0, 0)),
            pl.BlockSpec((1, KP), lambda i: (0, 0)),
            pl.BlockSpec((1, KP), lambda i: (0, 0)),
            pl.BlockSpec((1, KP), lambda i: (0, 0)),
        ],
        out_specs=pl.BlockSpec((BM, KP), lambda i: (i, 0)),
        out_shape=jax.ShapeDtypeStruct((NPAD, KP), jnp.bfloat16),
    )(signal_p, Wd, b, s, bt)


def _conv_layer(interp3, Tm, btile, s, bt):
    BM = 576

    def body(a_ref, t_ref, b_ref, s_ref, bt_ref, o_ref, acc):
        ra = pl.program_id(1)

        @pl.when(ra == 0)
        def _():
            acc[...] = jnp.broadcast_to(b_ref[...], (BM, NROT * KP))

        acc[...] += jnp.dot(a_ref[0].astype(jnp.bfloat16),
                            t_ref[...].astype(jnp.bfloat16),
                            preferred_element_type=jnp.float32)

        @pl.when(ra == RA - 1)
        def _():
            y = jnp.maximum(acc[...], 0.0)
            m = jnp.maximum(jnp.maximum(y[:, :KP], y[:, KP:2 * KP]),
                            y[:, 2 * KP:3 * KP])
            o_ref[...] = (m * s_ref[...] + bt_ref[...]).astype(jnp.bfloat16)

    return pl.pallas_call(
        body,
        grid=(NPAD // BM, RA),
        in_specs=[
            pl.BlockSpec((1, BM, KP), lambda i, ra: (ra, i, 0)),
            pl.BlockSpec((KP, NROT * KP), lambda i, ra: (ra, 0)),
            pl.BlockSpec((1, NROT * KP), lambda i, ra: (0, 0)),
            pl.BlockSpec((1, KP), lambda i, ra: (0, 0)),
            pl.BlockSpec((1, KP), lambda i, ra: (0, 0)),
        ],
        out_specs=pl.BlockSpec((BM, KP), lambda i, ra: (i, 0)),
        out_shape=jax.ShapeDtypeStruct((NPAD, KP), jnp.bfloat16),
        scratch_shapes=[pltpu.VMEM((BM, NROT * KP), jnp.float32)],
    )(interp3, Tm, btile, s, bt)


def _final_proj(x, Wo, bo):
    BM, BN = 512, 1024

    def body(a_ref, w_ref, b_ref, o_ref):
        o_ref[...] = jnp.dot(a_ref[...], w_ref[...].astype(jnp.bfloat16),
                             preferred_element_type=jnp.float32) + b_ref[...]

    return pl.pallas_call(
        body,
        grid=(pl.cdiv(NV, BM), pl.cdiv(NV, BN)),
        in_specs=[
            pl.BlockSpec((BM, KP), lambda i, j: (i, 0)),
            pl.BlockSpec((KP, BN), lambda i, j: (0, j)),
            pl.BlockSpec((1, BN), lambda i, j: (0, j)),
        ],
        out_specs=pl.BlockSpec((BM, BN), lambda i, j: (i, j)),
        out_shape=jax.ShapeDtypeStruct((NV, NV), jnp.float32),
    )(x, Wo, bo)


def _rot_templates(tpl, rd, kp, dp):
    K, Rq, Aq, D = tpl.shape
    tpl = jnp.pad(tpl, ((0, kp - K), (0, 0), (0, 0), (0, dp - D)))
    blk = np.concatenate([np.arange(0, 32, 2), np.arange(1, 32, 2)])
    d_of_c = (np.arange(dp).reshape(-1, 32)[:, blk]).reshape(-1)
    tpl = tpl[:, :, :, d_of_c]
    mats = []
    for rot in range(0, Aq, rd):
        t = jnp.roll(tpl, -rot, axis=2)
        mats.append(t.transpose(1, 2, 3, 0).reshape(Rq * Aq * dp, kp))
    return jnp.concatenate(mats, axis=1)


def _pad1(v, n):
    return jnp.pad(v, (0, n - v.shape[0]))


def kernel(signal, bc, W_down, b_down, gamma_down, beta_down,
           templates_0, bias_0, gamma_0, beta_0,
           templates_1, bias_1, gamma_1, beta_1,
           templates_2, bias_2, gamma_2, beta_2,
           W_out, b_out):
    bc_idx = bc[..., 0].astype(jnp.int32).reshape(NV, RA, 3)
    bc_w = bc[..., 1].reshape(NV, RA, 3)
    bc_idx = jnp.pad(bc_idx.transpose(1, 0, 2), ((0, 0), (0, NPAD - NV), (0, 0)))
    bc_w = jnp.pad(bc_w.transpose(1, 0, 2), ((0, 0), (0, NPAD - NV), (0, 0)))
    bc_idx = bc_idx.reshape(MPAD, 3)
    bc_w = bc_w.reshape(MPAD, 3)
    i0, i1, i2 = bc_idx[:, 0], bc_idx[:, 1], bc_idx[:, 2]
    w0, w1, w2 = bc_w[:, 0], bc_w[:, 1], bc_w[:, 2]

    x = _down_proj(signal, jnp.pad(W_down, ((0, 0), (0, KP - DOWN_D))),
                   _pad1(b_down, KP).reshape(1, KP),
                   (_pad1(gamma_down, KP) * INV_S).reshape(1, KP),
                   _pad1(beta_down, KP).reshape(1, KP))

    layers = (
        (templates_0, bias_0, gamma_0, beta_0, 2),
        (templates_1, bias_1, gamma_1, beta_1, 2),
        (templates_2, bias_2, gamma_2, beta_2, 2),
    )
    for tpl, b, g, bt, rd in layers:
        Tm = _rot_templates(tpl, rd, KP, KP)
        btile = jnp.tile(_pad1(b, KP), NROT).reshape(1, NROT * KP)
        interp = _gather128(x, i0, i1, i2, w0, w1, w2)
        interp3 = interp.reshape(RA, NPAD, KP)
        x = _conv_layer(interp3, Tm, btile,
                        (_pad1(g, KP) * INV_S).reshape(1, KP),
                        _pad1(bt, KP).reshape(1, KP))

    return _final_proj(x, W_out, b_out.reshape(1, NV))


# speedup vs baseline: 4.3194x; 1.2625x over previous
import functools

import numpy as np
import jax
import jax.numpy as jnp
from jax import lax
from jax.experimental import pallas as pl
from jax.experimental.pallas import tpu as pltpu
from jax.experimental.pallas import tpu_sc as plsc

NV = 6890
NPAD = 6912
RR, AA = 3, 6
RA = RR * AA
RAP = 24
MPAD = NPAD * RA
M24 = NPAD * RAP
NW = 32
ROWS_W = MPAD // NW
NV_W = NPAD // NW
CV = 4
CHUNK = CV * RA
CROWS = CV * RAP
NCH = NV_W // CV
RING = 3
OUTER = NCH // RING
INV_S = float(1.0 / np.sqrt(1.0 + 1e-3))

SIG_D = 544
DOWN_D = 64
KP = 128
NROT = 3
BCL = 128

_SC_PARAMS = pltpu.CompilerParams(use_tc_tiling_on_sc=False,
                                  needs_layout_passes=False)


def _make_format():
    mesh = plsc.VectorSubcoreMesh(core_axis_name="c", subcore_axis_name="s")

    def body(bc_hbm, i0h, i1h, i2h, w0h, w1h, w2h,
             buf, si0, si1, si2, sw0, sw1, sw2):
        wid = lax.axis_index("s") * 2 + lax.axis_index("c")
        pltpu.sync_copy(bc_hbm.at[pl.ds(wid * NV_W, NV_W)], buf)
        iota = lax.iota(jnp.int32, 16)
        si = (si0, si1, si2)
        sw = (sw0, sw1, sw2)
        for j in range(3):

            def grp(q, carry, j=j):
                mb = q * 16
                mv = mb + iota
                rl = mv // RA
                col = (mv % RA) * 6 + (2 * j)
                iv = plsc.load_gather(buf, [rl, col])
                wv = plsc.load_gather(buf, [rl, col + 1])
                si[j][pl.ds(mb, 16)] = iv.astype(jnp.int32)
                sw[j][pl.ds(mb, 16)] = wv
                return carry

            lax.fori_loop(0, ROWS_W // 16, grp, 0)
        base = wid * ROWS_W
        for h, v in ((i0h, si0), (i1h, si1), (i2h, si2),
                     (w0h, sw0), (w1h, sw1), (w2h, sw2)):
            pltpu.sync_copy(v, h.at[pl.ds(base, ROWS_W)])

    return pl.kernel(
        body, mesh=mesh,
        out_type=[jax.ShapeDtypeStruct((MPAD,), jnp.int32)] * 3
        + [jax.ShapeDtypeStruct((MPAD,), jnp.float32)] * 3,
        scratch_types=(
            [pltpu.VMEM((NV_W, BCL), jnp.float32)]
            + [pltpu.VMEM((ROWS_W,), jnp.int32) for _ in range(3)]
            + [pltpu.VMEM((ROWS_W,), jnp.float32) for _ in range(3)]
        ),
        compiler_params=_SC_PARAMS,
    )


_format_bc = _make_format()

_PERM = [(r // RA) * RAP + (r % RA) for r in range(CHUNK)]
_RBASES = [0, 16, 32, 48, 56]


def _make_gather(D):
    mesh = plsc.VectorSubcoreMesh(core_axis_name="c", subcore_axis_name="s")

    def body(x_hbm, i0h, i1h, i2h, w0h, w1h, w2h, out_hbm, *refs):
        (i0, i1, i2, w0, w1, w2,
         ga0, gb0, gc0, ga1, gb1, gc1, ga2, gb2, gc2,
         ov0, ov1, ov2,
         gs0, gs1, gs2, os0, os1, os2) = refs
        G = ((ga0, gb0, gc0), (ga1, gb1, gc1), (ga2, gb2, gc2))
        ov = (ov0, ov1, ov2)
        GS = (gs0, gs1, gs2)
        OS = (os0, os1, os2)
        idx = (i0, i1, i2)
        wts = (w0, w1, w2)
        wid = lax.axis_index("s") * 2 + lax.axis_index("c")
        base = wid * ROWS_W
        for h, v in ((i0h, i0), (i1h, i1), (i2h, i2),
                     (w0h, w0), (w1h, w1), (w2h, w2)):
            pltpu.sync_copy(h.at[pl.ds(base, ROWS_W)], v)
        zeros16 = jnp.zeros((16,), jnp.float32)
        for s in range(RING):
            for v in range(CV):
                for z in range(RAP - RA):
                    for db in range(D // 16):
                        ov[s][v * RAP + RA + z, pl.ds(db * 16, 16)] = zeros16

        def issue_gather(c, s):
            start = c * CHUNK
            for j in range(3):
                pltpu.async_copy(
                    x_hbm.at[idx[j].at[pl.ds(start, CHUNK)]], G[s][j], GS[s])

        def wait_gather(s):
            for j in range(3):
                pltpu.make_async_copy(
                    x_hbm.at[pl.ds(0, CHUNK)], G[s][j], GS[s]).wait()

        def issue_out(c, s):
            g = wid * NCH + c
            pltpu.async_copy(ov[s], out_hbm.at[pl.ds(g * CROWS, CROWS)], OS[s])

        def wait_out(s):
            pltpu.make_async_copy(
                ov[s], out_hbm.at[pl.ds(0, CROWS)], OS[s]).wait()

        def compute(c, s):
            ga, gb, gc = G[s]
            start = c * CHUNK
            for rb in _RBASES:
                wv0 = wts[0][pl.ds(start + rb, 16)]
                wv1 = wts[1][pl.ds(start + rb, 16)]
                wv2 = wts[2][pl.ds(start + rb, 16)]
                for i in range(16):
                    a, b2, c2 = wv0[i], wv1[i], wv2[i]
                    r = rb + i
                    p = _PERM[r]
                    for db in range(D // 32):
                        sdb = pl.ds(db * 32, 32)
                        a0, a1 = plsc.unpack(
                            ga[r, sdb], format=plsc.PackFormat.INTERLEAVED)
                        b0, b1 = plsc.unpack(
                            gb[r, sdb], format=plsc.PackFormat.INTERLEAVED)
                        c0, c1 = plsc.unpack(
                            gc[r, sdb], format=plsc.PackFormat.INTERLEAVED)
                        ov[s][p, pl.ds(db * 32, 16)] = (
                            a0 * a + b0 * b2 + c0 * c2)
                        ov[s][p, pl.ds(db * 32 + 16, 16)] = (
                            a1 * a + b1 * b2 + c1 * c2)

        issue_gather(0, 0)
        issue_gather(1, 1)

        def outer(k, carry):
            for b in range(RING):
                c = 3 * k + b
                sn = (b + 2) % 3
                if b == 0:
                    @pl.when(k >= 1)
                    def _():
                        wait_out(sn)
                    issue_gather(c + 2, sn)
                else:
                    wait_out(sn)

                    @pl.when(k <= OUTER - 2)
                    def _():
                        issue_gather(c + 2, sn)
                wait_gather(b)
                compute(c, b)
                issue_out(c, b)
            return carry

        lax.fori_loop(0, OUTER, outer, 0)
        wait_out(2)

    return pl.kernel(
        body, mesh=mesh,
        out_type=jax.ShapeDtypeStruct((M24, D), jnp.float32),
        scratch_types=(
            [pltpu.VMEM((ROWS_W,), jnp.int32) for _ in range(3)]
            + [pltpu.VMEM((ROWS_W,), jnp.float32) for _ in range(3)]
            + [pltpu.VMEM((CHUNK, D), jnp.bfloat16) for _ in range(9)]
            + [pltpu.VMEM((CROWS, D), jnp.float32) for _ in range(3)]
            + [pltpu.SemaphoreType.DMA for _ in range(6)]
        ),
        compiler_params=_SC_PARAMS,
    )


_gather128 = _make_gather(KP)


def _down_proj(signal, Wd, b, s, bt):
    BM = 576

    def body(a_ref, w_ref, b_ref, s_ref, bt_ref, o_ref):
        y = jnp.dot(a_ref[...], w_ref[...], preferred_element_type=jnp.float32)
        y = jnp.maximum(y + b_ref[...], 0.0)
        o_ref[...] = (y * s_ref[...] + bt_ref[...]).astype(jnp.bfloat16)

    return pl.pallas_call(
        body,
        grid=(NPAD // BM,),
        in_specs=[
            pl.BlockSpec((BM, SIG_D), lambda i: (i, 0)),
            pl.BlockSpec((SIG_D, KP), lambda i: (0, 0)),
            pl.BlockSpec((1, KP), lambda i: (0, 0)),
            pl.BlockSpec((1, KP), lambda i: (0, 0)),
            pl.BlockSpec((1, KP), lambda i: (0, 0)),
        ],
        out_specs=pl.BlockSpec((BM, KP), lambda i: (i, 0)),
        out_shape=jax.ShapeDtypeStruct((NPAD, KP), jnp.bfloat16),
    )(signal, Wd, b, s, bt)


def _conv_layer(interp24, Tm, btile, s, bt):
    BM = 576
    RB = BM * RAP

    def body(a_ref, t_ref, b_ref, s_ref, bt_ref, o_ref):
        a = a_ref[...].reshape(BM, RAP * KP)
        y = jnp.dot(a.astype(jnp.bfloat16), t_ref[...],
                    preferred_element_type=jnp.float32)
        y = jnp.maximum(y + b_ref[...], 0.0)
        m = jnp.maximum(jnp.maximum(y[:, :KP], y[:, KP:2 * KP]),
                        y[:, 2 * KP:3 * KP])
        o_ref[...] = (m * s_ref[...] + bt_ref[...]).astype(jnp.bfloat16)

    return pl.pallas_call(
        body,
        grid=(NPAD // BM,),
        in_specs=[
            pl.BlockSpec((RB, KP), lambda i: (i, 0)),
            pl.BlockSpec((RAP * KP, NROT * KP), lambda i: (0, 0)),
            pl.BlockSpec((1, NROT * KP), lambda i: (0, 0)),
            pl.BlockSpec((1, KP), lambda i: (0, 0)),
            pl.BlockSpec((1, KP), lambda i: (0, 0)),
        ],
        out_specs=pl.BlockSpec((BM, KP), lambda i: (i, 0)),
        out_shape=jax.ShapeDtypeStruct((NPAD, KP), jnp.bfloat16),
    )(interp24, Tm, btile, s, bt)


def _final_proj(x, Wo, bo):
    BM, BN = 512, 1024

    def body(a_ref, w_ref, b_ref, o_ref):
        o_ref[...] = jnp.dot(a_ref[...], w_ref[...],
                             preferred_element_type=jnp.float32) + b_ref[...]

    return pl.pallas_call(
        body,
        grid=(pl.cdiv(NV, BM), pl.cdiv(NV, BN)),
        in_specs=[
            pl.BlockSpec((BM, KP), lambda i, j: (i, 0)),
            pl.BlockSpec((KP, BN), lambda i, j: (0, j)),
            pl.BlockSpec((1, BN), lambda i, j: (0, j)),
        ],
        out_specs=pl.BlockSpec((BM, BN), lambda i, j: (i, j)),
        out_shape=jax.ShapeDtypeStruct((NV, NV), jnp.float32),
    )(x, Wo, bo)


def _rot_templates(tpl, rd, kp, dp):
    K, Rq, Aq, D = tpl.shape
    tpl = jnp.pad(tpl, ((0, kp - K), (0, 0), (0, 0), (0, dp - D)))
    blk = np.concatenate([np.arange(0, 32, 2), np.arange(1, 32, 2)])
    d_of_c = (np.arange(dp).reshape(-1, 32)[:, blk]).reshape(-1)
    tpl = tpl[:, :, :, d_of_c]
    mats = []
    for rot in range(0, Aq, rd):
        t = jnp.roll(tpl, -rot, axis=2)
        mats.append(t.transpose(1, 2, 3, 0).reshape(Rq * Aq * dp, kp))
    Tm = jnp.concatenate(mats, axis=1)
    return jnp.pad(Tm, ((0, (RAP - RA) * dp), (0, 0))).astype(jnp.bfloat16)


def _pad1(v, n):
    return jnp.pad(v, (0, n - v.shape[0]))


def kernel(signal, bc, W_down, b_down, gamma_down, beta_down,
           templates_0, bias_0, gamma_0, beta_0,
           templates_1, bias_1, gamma_1, beta_1,
           templates_2, bias_2, gamma_2, beta_2,
           W_out, b_out):
    bcp = jnp.pad(bc.reshape(NV, RA * 6), ((0, NPAD - NV), (0, BCL - RA * 6)))
    i0, i1, i2, w0, w1, w2 = _format_bc(bcp)

    x = _down_proj(signal, jnp.pad(W_down, ((0, 0), (0, KP - DOWN_D))),
                   _pad1(b_down, KP).reshape(1, KP),
                   (_pad1(gamma_down, KP) * INV_S).reshape(1, KP),
                   _pad1(beta_down, KP).reshape(1, KP))

    layers = (
        (templates_0, bias_0, gamma_0, beta_0, 2),
        (templates_1, bias_1, gamma_1, beta_1, 2),
        (templates_2, bias_2, gamma_2, beta_2, 2),
    )
    for tpl, b, g, bt, rd in layers:
        Tm = _rot_templates(tpl, rd, KP, KP)
        btile = jnp.tile(_pad1(b, KP), NROT).reshape(1, NROT * KP)
        interp24 = _gather128(x, i0, i1, i2, w0, w1, w2)
        x = _conv_layer(interp24, Tm, btile,
                        (_pad1(g, KP) * INV_S).reshape(1, KP),
                        _pad1(bt, KP).reshape(1, KP))

    return _final_proj(x, W_out.astype(jnp.bfloat16), b_out.reshape(1, NV))
